# baseline (speedup 1.0000x reference)
import jax
import jax.numpy as jnp
from jax.experimental import pallas as pl
from jax.experimental.pallas import tpu as pltpu

B, N, T = 32, 1024, 48
BT = B * T
NBLK = 1
R = N // NBLK

LAMBDA_PHYS = 0.1
LAMBDA_SMOOTH = 0.01


def _body(x2_ref, t2blk_ref, r2blk_ref, adj_ref,
          tot_ref, pl_ref, ph_ref, sm_ref):
    i = pl.program_id(0)

    x2 = x2_ref[...]
    qrow = jnp.sum(x2 * x2, axis=0, keepdims=True)

    xb = x2
    dp = xb - t2blk_ref[...]
    rr = r2blk_ref[...]

    xbf = xb.astype(jnp.bfloat16)
    c = jax.lax.dot_general(
        xbf, xbf, (((0,), (0,)), ((), ())),
        preferred_element_type=jnp.float32)

    r0 = i * R
    rows = r0 + jax.lax.broadcasted_iota(jnp.int32, (R, N), 0)
    cols = jax.lax.broadcasted_iota(jnp.int32, (R, N), 1)
    a = adj_ref[...]
    w = jnp.where(cols > rows, a, 0.0)
    qcol = jnp.sum(jnp.where(cols == rows, c, 0.0), axis=1, keepdims=True)

    t2s = jnp.sum(w * c)
    t1s = jnp.sum(w * (qcol + qrow))

    denom = float(B * N * T)
    pred_loss = jnp.sum(dp * dp) / denom
    physics_loss = jnp.sum(rr * rr) / denom
    smooth_loss = (t1s - 2.0 * t2s) / denom
    total = (pred_loss + LAMBDA_PHYS * physics_loss
             + LAMBDA_SMOOTH * smooth_loss)
    tot_ref[0, 0] = total
    pl_ref[0, 0] = pred_loss
    ph_ref[0, 0] = physics_loss
    sm_ref[0, 0] = smooth_loss


def _pallas(x2, t2, r2, adj, *, interpret=False):
    f32 = jnp.float32
    return pl.pallas_call(
        _body,
        grid=(NBLK,),
        in_specs=[
            pl.BlockSpec((BT, N), lambda i: (0, 0)),
            pl.BlockSpec((BT, R), lambda i: (0, i)),
            pl.BlockSpec((BT, R), lambda i: (0, i)),
            pl.BlockSpec((R, N), lambda i: (i, 0)),
        ],
        out_specs=[
            pl.BlockSpec(memory_space=pltpu.SMEM),
            pl.BlockSpec(memory_space=pltpu.SMEM),
            pl.BlockSpec(memory_space=pltpu.SMEM),
            pl.BlockSpec(memory_space=pltpu.SMEM),
        ],
        out_shape=[
            jax.ShapeDtypeStruct((1, 1), f32),
            jax.ShapeDtypeStruct((1, 1), f32),
            jax.ShapeDtypeStruct((1, 1), f32),
            jax.ShapeDtypeStruct((1, 1), f32),
        ],
        compiler_params=pltpu.CompilerParams(
            dimension_semantics=("parallel",),
            vmem_limit_bytes=50 * 1024 * 1024,
        ),
        name="physics_loss",
        interpret=interpret,
    )(x2, t2, r2, adj)


def kernel(predictions, targets, physics_residuals, adj, *, interpret=False):
    x2 = predictions.transpose(0, 2, 1).reshape(BT, N)
    t2 = targets.transpose(0, 2, 1).reshape(BT, N)
    r2 = physics_residuals.transpose(0, 2, 1).reshape(BT, N)
    tot, pls, phs, sms = _pallas(x2, t2, r2, adj, interpret=interpret)
    return (tot.reshape(()), pls.reshape(()), phs.reshape(()),
            sms.reshape(()))


# speedup vs baseline: 1.0157x; 1.0157x over previous
import jax
import jax.numpy as jnp
from jax.experimental import pallas as pl
from jax.experimental.pallas import tpu as pltpu

B, N, T = 32, 1024, 48
BT = B * T
NBLK = 1
R = N // NBLK

LAMBDA_PHYS = 0.1
LAMBDA_SMOOTH = 0.01


def _body(x2_ref, t2blk_ref, r2blk_ref, adj_ref,
          tot_ref, pl_ref, ph_ref, sm_ref):
    i = pl.program_id(0)

    x2 = x2_ref[...]
    qrow = jnp.sum(x2 * x2, axis=0, keepdims=True)

    xb = x2
    dp = xb - t2blk_ref[...]
    rr = r2blk_ref[...]

    c = jax.lax.dot_general(
        xb, x2, (((0,), (0,)), ((), ())),
        preferred_element_type=jnp.float32)

    r0 = i * R
    rows = r0 + jax.lax.broadcasted_iota(jnp.int32, (R, N), 0)
    cols = jax.lax.broadcasted_iota(jnp.int32, (R, N), 1)
    a = adj_ref[...]
    w = jnp.where(cols > rows, a, 0.0)
    qcol = jnp.sum(jnp.where(cols == rows, c, 0.0), axis=1, keepdims=True)

    t2s = jnp.sum(w * c)
    t1s = jnp.sum(w * (qcol + qrow))

    denom = float(B * N * T)
    pred_loss = jnp.sum(dp * dp) / denom
    physics_loss = jnp.sum(rr * rr) / denom
    smooth_loss = (t1s - 2.0 * t2s) / denom
    total = (pred_loss + LAMBDA_PHYS * physics_loss
             + LAMBDA_SMOOTH * smooth_loss)
    tot_ref[0, 0] = total
    pl_ref[0, 0] = pred_loss
    ph_ref[0, 0] = physics_loss
    sm_ref[0, 0] = smooth_loss


def _pallas(x2, t2, r2, adj, *, interpret=False):
    f32 = jnp.float32
    return pl.pallas_call(
        _body,
        grid=(NBLK,),
        in_specs=[
            pl.BlockSpec((BT, N), lambda i: (0, 0)),
            pl.BlockSpec((BT, R), lambda i: (0, i)),
            pl.BlockSpec((BT, R), lambda i: (0, i)),
            pl.BlockSpec((R, N), lambda i: (i, 0)),
        ],
        out_specs=[
            pl.BlockSpec(memory_space=pltpu.SMEM),
            pl.BlockSpec(memory_space=pltpu.SMEM),
            pl.BlockSpec(memory_space=pltpu.SMEM),
            pl.BlockSpec(memory_space=pltpu.SMEM),
        ],
        out_shape=[
            jax.ShapeDtypeStruct((1, 1), f32),
            jax.ShapeDtypeStruct((1, 1), f32),
            jax.ShapeDtypeStruct((1, 1), f32),
            jax.ShapeDtypeStruct((1, 1), f32),
        ],
        compiler_params=pltpu.CompilerParams(
            dimension_semantics=("parallel",),
            vmem_limit_bytes=50 * 1024 * 1024,
        ),
        name="physics_loss",
        interpret=interpret,
    )(x2, t2, r2, adj)


def kernel(predictions, targets, physics_residuals, adj, *, interpret=False):
    x2 = predictions.transpose(0, 2, 1).reshape(BT, N)
    t2 = targets.transpose(0, 2, 1).reshape(BT, N)
    r2 = physics_residuals.transpose(0, 2, 1).reshape(BT, N)
    tot, pls, phs, sms = _pallas(x2, t2, r2, adj, interpret=interpret)
    return (tot.reshape(()), pls.reshape(()), phs.reshape(()),
            sms.reshape(()))
